# baseline (speedup 1.0000x reference)
import functools

import jax
import jax.numpy as jnp
from jax import lax
from jax.experimental import pallas as pl
from jax.experimental.pallas import tpu as pltpu
from jax.experimental.pallas import tpu_sc as plsc

_B, _S, _D = 1024, 200, 128
_E = 32
_N = _B * _S

_CG = 640


def _ring(nbuf, steps, load, store):
    ld = [None] * nbuf
    st = [None] * nbuf

    def prime():
        for b in range(min(nbuf, steps)):
            ld[b] = load(b)

    def step(t):
        b = t % nbuf
        if t >= 1 and t + nbuf - 1 < steps:
            st[(t - 1) % nbuf].wait()
            ld[(t - 1) % nbuf] = load(t + nbuf - 1)
        ld[b].wait()
        st[b] = store(t)

    def drain():
        for t in range(max(steps - nbuf, 0), steps):
            st[t % nbuf].wait()

    return prime, step, drain


def _sc_gather(ner1d, table):
    info = plsc.get_sparse_core_info()
    nw = info.num_cores * info.num_subcores
    n_per_w = _N // nw
    steps = n_per_w // _CG

    mesh = plsc.VectorSubcoreMesh(core_axis_name="c", subcore_axis_name="s")

    @functools.partial(
        pl.kernel,
        mesh=mesh,
        out_type=jax.ShapeDtypeStruct((_N, _E), jnp.float32),
        compiler_params=pltpu.CompilerParams(use_tc_tiling_on_sc=False),
        scratch_types=[
            pltpu.VMEM((2, _CG), jnp.int32),
            pltpu.VMEM((2, _CG, _E), jnp.float32),
            pltpu.SemaphoreType.DMA((2,)),
            pltpu.SemaphoreType.DMA((2,)),
            pltpu.SemaphoreType.DMA((2,)),
        ],
    )
    def k(ner_hbm, table_hbm, emb_hbm, idx, rows, sem_i, sem_g, sem_r):
        wid = lax.axis_index("s") * info.num_cores + lax.axis_index("c")
        base = wid * n_per_w

        def iload(s):
            b = s % 2
            return pltpu.async_copy(
                ner_hbm.at[pl.ds(base + s * _CG, _CG)], idx.at[b],
                sem_i.at[b])

        def gath(s):
            b = s % 2
            return pltpu.async_copy(table_hbm.at[idx.at[b]], rows.at[b],
                                    sem_g.at[b])

        def rstore(s):
            b = s % 2
            return pltpu.async_copy(
                rows.at[b],
                emb_hbm.at[pl.ds(base + s * _CG, _CG), :],
                sem_r.at[b])

        gi = [None] * 2
        gg = [None] * 2
        gr = [None] * 2

        gi[0] = iload(0)
        gi[1] = iload(1)
        for s in range(steps):
            b = s % 2
            if s >= 1:
                gg[1 - b].wait()
                gr[1 - b] = rstore(s - 1)
                if s + 1 < steps:
                    gi[1 - b] = iload(s + 1)
            if s >= 2:
                gr[b].wait()
            gi[b].wait()
            gg[b] = gath(s)
        gg[(steps - 1) % 2].wait()
        gr[(steps - 1) % 2] = rstore(steps - 1)
        gr[steps % 2].wait()
        gr[(steps - 1) % 2].wait()

    return k(ner1d, table)


def _sc_concat(x3, emb):
    info = plsc.get_sparse_core_info()
    nw = info.num_cores * info.num_subcores
    b_per_w = _B // nw

    mesh = plsc.VectorSubcoreMesh(core_axis_name="c", subcore_axis_name="s")

    @functools.partial(
        pl.kernel,
        mesh=mesh,
        out_type=jax.ShapeDtypeStruct((_B, _S, _D + _E), jnp.float32),
        scratch_types=[
            pltpu.VMEM((2, 1, _S, _D), jnp.float32),
            pltpu.VMEM((2, 1, _S, _E), jnp.float32),
            pltpu.SemaphoreType.DMA((2,)),
            pltpu.SemaphoreType.DMA((2,)),
            pltpu.SemaphoreType.DMA((2,)),
            pltpu.SemaphoreType.DMA((2,)),
        ],
    )
    def k(x_hbm, emb_hbm, out_hbm, xbuf, ebuf,
          sem_xl, sem_xs, sem_el, sem_es):
        wid = lax.axis_index("s") * info.num_cores + lax.axis_index("c")
        b0 = wid * b_per_w

        xp, xstep, xdrain = _ring(
            2, b_per_w,
            lambda t: pltpu.async_copy(
                x_hbm.at[pl.ds(b0 + t, 1)], xbuf.at[t % 2], sem_xl.at[t % 2]),
            lambda t: pltpu.async_copy(
                xbuf.at[t % 2],
                out_hbm.at[pl.ds(b0 + t, 1), :, pl.ds(0, _D)],
                sem_xs.at[t % 2]),
        )
        ep, estep, edrain = _ring(
            2, b_per_w,
            lambda t: pltpu.async_copy(
                emb_hbm.at[pl.ds((b0 + t) * _S, _S), :],
                ebuf.at[t % 2, 0], sem_el.at[t % 2]),
            lambda t: pltpu.async_copy(
                ebuf.at[t % 2],
                out_hbm.at[pl.ds(b0 + t, 1), :, pl.ds(_D, _E)],
                sem_es.at[t % 2]),
        )

        xp()
        ep()
        for t in range(b_per_w):
            estep(t)
            xstep(t)
        xdrain()
        edrain()

    return k(x3, emb)


def kernel(x, ner, pos, entity_table):
    del pos
    ner1d = ner.reshape(_N).astype(jnp.int32)
    emb = _sc_gather(ner1d, entity_table)
    return _sc_concat(x, emb)


# speedup vs baseline: 1.5844x; 1.5844x over previous
import functools

import jax
import jax.numpy as jnp
from jax import lax
from jax.experimental import pallas as pl
from jax.experimental.pallas import tpu as pltpu
from jax.experimental.pallas import tpu_sc as plsc

_B, _S, _D = 1024, 200, 128
_E = 32
_N = _B * _S


def _sc_gather(ner2, table):
    info = plsc.get_sparse_core_info()
    nw = info.num_cores * info.num_subcores
    b_per_w = _B // nw

    mesh = plsc.VectorSubcoreMesh(core_axis_name="c", subcore_axis_name="s")

    @functools.partial(
        pl.kernel,
        mesh=mesh,
        out_type=jax.ShapeDtypeStruct((_B, _S, _D), jnp.float32),
        compiler_params=pltpu.CompilerParams(use_tc_tiling_on_sc=False),
        scratch_types=[
            pltpu.VMEM((2, 1, _S), jnp.int32),
            pltpu.VMEM((2, _S, _E), jnp.float32),
            pltpu.SemaphoreType.DMA((2,)),
            pltpu.SemaphoreType.DMA((2,)),
            pltpu.SemaphoreType.DMA((2,)),
        ],
    )
    def k(ner_hbm, table_hbm, emb_hbm, idx, rows, sem_i, sem_g, sem_r):
        wid = lax.axis_index("s") * info.num_cores + lax.axis_index("c")
        b0 = wid * b_per_w

        def iload(s):
            b = s % 2
            return pltpu.async_copy(
                ner_hbm.at[pl.ds(b0 + s, 1), :], idx.at[b], sem_i.at[b])

        def gath(s):
            b = s % 2
            return pltpu.async_copy(table_hbm.at[idx.at[b, 0]], rows.at[b],
                                    sem_g.at[b])

        def rstore(s):
            b = s % 2
            return pltpu.async_copy(
                rows.at[b],
                emb_hbm.at[b0 + s, :, pl.ds(0, _E)],
                sem_r.at[b])

        gi = [None] * 2
        gg = [None] * 2
        gr = [None] * 2

        gi[0] = iload(0)
        gi[1] = iload(1)
        for s in range(b_per_w):
            b = s % 2
            if s >= 1:
                gg[1 - b].wait()
                gr[1 - b] = rstore(s - 1)
                if s + 1 < b_per_w:
                    gi[1 - b] = iload(s + 1)
            if s >= 2:
                gr[b].wait()
            gi[b].wait()
            gg[b] = gath(s)
        gg[(b_per_w - 1) % 2].wait()
        gr[(b_per_w - 1) % 2] = rstore(b_per_w - 1)
        gr[b_per_w % 2].wait()
        gr[(b_per_w - 1) % 2].wait()

    return k(ner2, table)


def kernel(x, ner, pos, entity_table):
    del pos
    emb_p = _sc_gather(ner.astype(jnp.int32), entity_table)
    return jnp.concatenate([x, emb_p[:, :, :_E]], axis=-1)
